# Initial kernel scaffold; baseline (speedup 1.0000x reference)
import jax
import jax.numpy as jnp
from jax.experimental import pallas as pl


def kernel(x, edge_index, adj, W1, b1, W2, b2):
    raise NotImplementedError("write your pallas kernel here")


# speedup vs baseline: 30.4866x; 30.4866x over previous
import functools

import jax
import jax.numpy as jnp
from jax import lax
from jax.experimental import pallas as pl
from jax.experimental.pallas import tpu as pltpu
from jax.experimental.pallas import tpu_sc as plsc

N = 10000
E = 320000
IN_CH = 128
HID = 64
K = 16

NC, NS = 2, 16
NW = NC * NS
EPT = E // NW
CH = 125
NCHUNK = EPT // CH
RSTRIDE = 624
RLEN = 640

@functools.cache
def _mesh():
    return plsc.VectorSubcoreMesh(
        core_axis_name="c", subcore_axis_name="s", num_cores=NC, num_subcores=NS)


@functools.cache
def _make_prop(D):
    @functools.partial(
        pl.kernel,
        out_type=jax.ShapeDtypeStruct((NC, N, D), jnp.float32),
        mesh=_mesh(),
        compiler_params=pltpu.CompilerParams(use_tc_tiling_on_sc=False),
        scratch_types=[
            pltpu.VMEM((NCHUNK, CH), jnp.int32),
            pltpu.VMEM((NCHUNK, CH), jnp.int32),
            pltpu.VMEM((CH, D), jnp.float32),
            pltpu.VMEM_SHARED((N, D), jnp.float32),
            pltpu.SemaphoreType.DMA,
        ],
    )
    def prop(edges, g, zeros, out, src_v, dst_v, rows_v, acc_sh, sem):
        c = lax.axis_index("c")
        s = lax.axis_index("s")
        wid = c * NS + s
        pltpu.sync_copy(edges.at[0, wid], src_v)
        pltpu.sync_copy(edges.at[1, wid], dst_v)
        r0 = s * RSTRIDE
        pltpu.sync_copy(zeros.at[pl.ds(r0, RLEN)], acc_sh.at[pl.ds(r0, RLEN)])
        plsc.subcore_barrier()

        def body(j, carry):
            pltpu.async_copy(g.at[src_v.at[j]], rows_v, sem).wait()
            pltpu.sync_copy(rows_v, acc_sh.at[dst_v.at[j]], add=True)
            return carry

        lax.fori_loop(0, NCHUNK, body, 0, unroll=False)
        plsc.subcore_barrier()
        pltpu.sync_copy(acc_sh.at[pl.ds(r0, RLEN)], out.at[c, pl.ds(r0, RLEN)])

    return prop


@functools.cache
def _make_deg():
    @functools.partial(
        pl.kernel,
        out_type=jax.ShapeDtypeStruct((NC, N, K), jnp.float32),
        mesh=_mesh(),
        compiler_params=pltpu.CompilerParams(use_tc_tiling_on_sc=False),
        scratch_types=[
            pltpu.VMEM((NCHUNK, CH), jnp.int32),
            pltpu.VMEM((CH, K), jnp.float32),
            pltpu.VMEM_SHARED((N, K), jnp.float32),
            pltpu.SemaphoreType.DMA,
        ],
    )
    def deg(edges, ones, zeros, out, dst_v, ones_v, acc_sh, sem):
        c = lax.axis_index("c")
        s = lax.axis_index("s")
        wid = c * NS + s
        pltpu.sync_copy(edges.at[1, wid], dst_v)
        pltpu.sync_copy(ones, ones_v)
        r0 = s * RSTRIDE
        pltpu.sync_copy(zeros.at[pl.ds(r0, RLEN)], acc_sh.at[pl.ds(r0, RLEN)])
        plsc.subcore_barrier()

        def body(j, carry):
            pltpu.sync_copy(ones_v, acc_sh.at[dst_v.at[j]], add=True)
            return carry

        lax.fori_loop(0, NCHUNK, body, 0, unroll=False)
        plsc.subcore_barrier()
        pltpu.sync_copy(acc_sh.at[pl.ds(r0, RLEN)], out.at[c, pl.ds(r0, RLEN)])

    return deg


RB = 1000


def _u_of(degp0, degp1):
    deg = degp0[:, 0] + degp1[:, 0] + 1.0
    return lax.rsqrt(deg)


def _g1_body(x_ref, w_ref, degp_ref, g1_ref):
    u = _u_of(degp_ref[0], degp_ref[1])
    hw = jnp.dot(x_ref[...], w_ref[...], preferred_element_type=jnp.float32,
                 precision=lax.Precision.HIGHEST)
    g1_ref[...] = hw * u[:, None]


def _mid_body(s1p_ref, g1_ref, degp_ref, b1_ref, w2_ref, g2_ref):
    u = _u_of(degp_ref[0], degp_ref[1])
    h1 = u[:, None] * (s1p_ref[0] + s1p_ref[1] + g1_ref[...]) + b1_ref[...]
    h1 = jnp.maximum(h1, 0.0)
    hw2 = jnp.dot(h1, w2_ref[...], preferred_element_type=jnp.float32,
                  precision=lax.Precision.HIGHEST)
    g2_ref[...] = hw2 * u[:, None]


def _out_body(s2p_ref, g2_ref, degp_ref, b2_ref, o_ref):
    u = _u_of(degp_ref[0], degp_ref[1])
    logits = u[:, None] * (s2p_ref[0] + s2p_ref[1] + g2_ref[...]) + b2_ref[...]
    m = jnp.max(logits, axis=1, keepdims=True)
    e = jnp.exp(logits - m)
    o_ref[...] = e / jnp.sum(e, axis=1, keepdims=True)


def kernel(x, edge_index, adj, W1, b1, W2, b2):
    del adj
    er = edge_index.astype(jnp.int32).reshape(2, NW, NCHUNK, CH)
    z64 = jnp.zeros((N, HID), jnp.float32)
    z16 = jnp.zeros((N, K), jnp.float32)
    ones = jnp.ones((CH, K), jnp.float32)

    degp = _make_deg()(er, ones, z16)

    grid = (N // RB,)
    g1 = pl.pallas_call(
        _g1_body,
        grid=grid,
        in_specs=[
            pl.BlockSpec((RB, IN_CH), lambda i: (i, 0)),
            pl.BlockSpec((IN_CH, HID), lambda i: (0, 0)),
            pl.BlockSpec((NC, RB, K), lambda i: (0, i, 0)),
        ],
        out_specs=pl.BlockSpec((RB, HID), lambda i: (i, 0)),
        out_shape=jax.ShapeDtypeStruct((N, HID), jnp.float32),
    )(x, W1, degp)

    s1p = _make_prop(HID)(er, g1, z64)

    g2 = pl.pallas_call(
        _mid_body,
        grid=grid,
        in_specs=[
            pl.BlockSpec((NC, RB, HID), lambda i: (0, i, 0)),
            pl.BlockSpec((RB, HID), lambda i: (i, 0)),
            pl.BlockSpec((NC, RB, K), lambda i: (0, i, 0)),
            pl.BlockSpec((HID,), lambda i: (0,)),
            pl.BlockSpec((HID, K), lambda i: (0, 0)),
        ],
        out_specs=pl.BlockSpec((RB, K), lambda i: (i, 0)),
        out_shape=jax.ShapeDtypeStruct((N, K), jnp.float32),
    )(s1p, g1, degp, b1, W2)

    s2p = _make_prop(K)(er, g2, z16)

    out = pl.pallas_call(
        _out_body,
        grid=grid,
        in_specs=[
            pl.BlockSpec((NC, RB, K), lambda i: (0, i, 0)),
            pl.BlockSpec((RB, K), lambda i: (i, 0)),
            pl.BlockSpec((NC, RB, K), lambda i: (0, i, 0)),
            pl.BlockSpec((K,), lambda i: (0,)),
        ],
        out_specs=pl.BlockSpec((RB, K), lambda i: (i, 0)),
        out_shape=jax.ShapeDtypeStruct((N, K), jnp.float32),
    )(s2p, g2, degp, b2)
    return out


# speedup vs baseline: 47.4109x; 1.5551x over previous
import functools

import jax
import jax.numpy as jnp
from jax import lax
from jax.experimental import pallas as pl
from jax.experimental.pallas import tpu as pltpu
from jax.experimental.pallas import tpu_sc as plsc

N = 10000
E = 320000
IN_CH = 128
HID = 64
K = 16

NC, NS = 2, 16
NW = NC * NS
EPT = E // NW
CH = 125
NCHUNK = EPT // CH
NBUF = 4
RSTRIDE = 624
RLEN = 640

@functools.cache
def _mesh():
    return plsc.VectorSubcoreMesh(
        core_axis_name="c", subcore_axis_name="s", num_cores=NC, num_subcores=NS)


@functools.cache
def _make_prop(D):
    @functools.partial(
        pl.kernel,
        out_type=jax.ShapeDtypeStruct((NC, N, D), jnp.float32),
        mesh=_mesh(),
        compiler_params=pltpu.CompilerParams(use_tc_tiling_on_sc=False),
        scratch_types=[
            pltpu.VMEM((NCHUNK, CH), jnp.int32),
            pltpu.VMEM((NCHUNK, CH), jnp.int32),
            pltpu.VMEM((NBUF, CH, D), jnp.float32),
            pltpu.VMEM_SHARED((N, D), jnp.float32),
            pltpu.SemaphoreType.DMA((NBUF,)),
        ],
    )
    def prop(edges, g, zeros, out, src_v, dst_v, rows_v, acc_sh, sems):
        c = lax.axis_index("c")
        s = lax.axis_index("s")
        wid = c * NS + s
        pltpu.sync_copy(edges.at[0, wid], src_v)
        pltpu.sync_copy(edges.at[1, wid], dst_v)
        r0 = s * RSTRIDE
        pltpu.sync_copy(zeros.at[pl.ds(r0, RLEN)], acc_sh.at[pl.ds(r0, RLEN)])
        plsc.subcore_barrier()

        for b in range(NBUF):
            pltpu.async_copy(g.at[src_v.at[b]], rows_v.at[b], sems.at[b])

        def body(j, carry):
            slot = lax.rem(j, NBUF)
            pltpu.make_async_copy(
                g.at[src_v.at[j]], rows_v.at[slot], sems.at[slot]).wait()
            pltpu.sync_copy(rows_v.at[slot], acc_sh.at[dst_v.at[j]], add=True)

            @pl.when(j + NBUF < NCHUNK)
            def _():
                pltpu.async_copy(
                    g.at[src_v.at[j + NBUF]], rows_v.at[slot], sems.at[slot])

            return carry

        lax.fori_loop(0, NCHUNK, body, 0, unroll=False)
        plsc.subcore_barrier()
        pltpu.sync_copy(acc_sh.at[pl.ds(r0, RLEN)], out.at[c, pl.ds(r0, RLEN)])

    return prop


@functools.cache
def _make_deg():
    @functools.partial(
        pl.kernel,
        out_type=jax.ShapeDtypeStruct((NC, N, K), jnp.float32),
        mesh=_mesh(),
        compiler_params=pltpu.CompilerParams(use_tc_tiling_on_sc=False),
        scratch_types=[
            pltpu.VMEM((NCHUNK, CH), jnp.int32),
            pltpu.VMEM((CH, K), jnp.float32),
            pltpu.VMEM_SHARED((N, K), jnp.float32),
            pltpu.SemaphoreType.DMA,
        ],
    )
    def deg(edges, ones, zeros, out, dst_v, ones_v, acc_sh, sem):
        c = lax.axis_index("c")
        s = lax.axis_index("s")
        wid = c * NS + s
        pltpu.sync_copy(edges.at[1, wid], dst_v)
        pltpu.sync_copy(ones, ones_v)
        r0 = s * RSTRIDE
        pltpu.sync_copy(zeros.at[pl.ds(r0, RLEN)], acc_sh.at[pl.ds(r0, RLEN)])
        plsc.subcore_barrier()

        def body(j, carry):
            pltpu.sync_copy(ones_v, acc_sh.at[dst_v.at[j]], add=True)
            return carry

        lax.fori_loop(0, NCHUNK, body, 0, unroll=False)
        plsc.subcore_barrier()
        pltpu.sync_copy(acc_sh.at[pl.ds(r0, RLEN)], out.at[c, pl.ds(r0, RLEN)])

    return deg


RB = 1000


def _u_of(degp0, degp1):
    deg = degp0[:, 0] + degp1[:, 0] + 1.0
    return lax.rsqrt(deg)


def _g1_body(x_ref, w_ref, degp_ref, g1_ref):
    u = _u_of(degp_ref[0], degp_ref[1])
    hw = jnp.dot(x_ref[...], w_ref[...], preferred_element_type=jnp.float32,
                 precision=lax.Precision.HIGHEST)
    g1_ref[...] = hw * u[:, None]


def _mid_body(s1p_ref, g1_ref, degp_ref, b1_ref, w2_ref, g2_ref):
    u = _u_of(degp_ref[0], degp_ref[1])
    h1 = u[:, None] * (s1p_ref[0] + s1p_ref[1] + g1_ref[...]) + b1_ref[...]
    h1 = jnp.maximum(h1, 0.0)
    hw2 = jnp.dot(h1, w2_ref[...], preferred_element_type=jnp.float32,
                  precision=lax.Precision.HIGHEST)
    g2_ref[...] = hw2 * u[:, None]


def _out_body(s2p_ref, g2_ref, degp_ref, b2_ref, o_ref):
    u = _u_of(degp_ref[0], degp_ref[1])
    logits = u[:, None] * (s2p_ref[0] + s2p_ref[1] + g2_ref[...]) + b2_ref[...]
    m = jnp.max(logits, axis=1, keepdims=True)
    e = jnp.exp(logits - m)
    o_ref[...] = e / jnp.sum(e, axis=1, keepdims=True)


def kernel(x, edge_index, adj, W1, b1, W2, b2):
    del adj
    er = edge_index.astype(jnp.int32).reshape(2, NW, NCHUNK, CH)
    z64 = jnp.zeros((N, HID), jnp.float32)
    z16 = jnp.zeros((N, K), jnp.float32)
    ones = jnp.ones((CH, K), jnp.float32)

    degp = _make_deg()(er, ones, z16)

    grid = (N // RB,)
    g1 = pl.pallas_call(
        _g1_body,
        grid=grid,
        in_specs=[
            pl.BlockSpec((RB, IN_CH), lambda i: (i, 0)),
            pl.BlockSpec((IN_CH, HID), lambda i: (0, 0)),
            pl.BlockSpec((NC, RB, K), lambda i: (0, i, 0)),
        ],
        out_specs=pl.BlockSpec((RB, HID), lambda i: (i, 0)),
        out_shape=jax.ShapeDtypeStruct((N, HID), jnp.float32),
    )(x, W1, degp)

    s1p = _make_prop(HID)(er, g1, z64)

    g2 = pl.pallas_call(
        _mid_body,
        grid=grid,
        in_specs=[
            pl.BlockSpec((NC, RB, HID), lambda i: (0, i, 0)),
            pl.BlockSpec((RB, HID), lambda i: (i, 0)),
            pl.BlockSpec((NC, RB, K), lambda i: (0, i, 0)),
            pl.BlockSpec((HID,), lambda i: (0,)),
            pl.BlockSpec((HID, K), lambda i: (0, 0)),
        ],
        out_specs=pl.BlockSpec((RB, K), lambda i: (i, 0)),
        out_shape=jax.ShapeDtypeStruct((N, K), jnp.float32),
    )(s1p, g1, degp, b1, W2)

    s2p = _make_prop(K)(er, g2, z16)

    out = pl.pallas_call(
        _out_body,
        grid=grid,
        in_specs=[
            pl.BlockSpec((NC, RB, K), lambda i: (0, i, 0)),
            pl.BlockSpec((RB, K), lambda i: (i, 0)),
            pl.BlockSpec((NC, RB, K), lambda i: (0, i, 0)),
            pl.BlockSpec((K,), lambda i: (0,)),
        ],
        out_specs=pl.BlockSpec((RB, K), lambda i: (i, 0)),
        out_shape=jax.ShapeDtypeStruct((N, K), jnp.float32),
    )(s2p, g2, degp, b2)
    return out


# speedup vs baseline: 50.7957x; 1.0714x over previous
import functools

import jax
import jax.numpy as jnp
from jax import lax
from jax.experimental import pallas as pl
from jax.experimental.pallas import tpu as pltpu
from jax.experimental.pallas import tpu_sc as plsc

N = 10000
E = 320000
IN_CH = 128
HID = 64
K = 16

NC, NS = 2, 16
NW = NC * NS
EPT = E // NW
CH = 125
NCHUNK = EPT // CH
NBUF = 8
GPRE = 4
DEGK = 8
RSTRIDE = 624
RLEN = 640

@functools.cache
def _mesh():
    return plsc.VectorSubcoreMesh(
        core_axis_name="c", subcore_axis_name="s", num_cores=NC, num_subcores=NS)


@functools.cache
def _make_prop(D):
    @functools.partial(
        pl.kernel,
        out_type=jax.ShapeDtypeStruct((NC, N, D), jnp.float32),
        mesh=_mesh(),
        compiler_params=pltpu.CompilerParams(use_tc_tiling_on_sc=False),
        scratch_types=[
            pltpu.VMEM((NCHUNK, CH), jnp.int32),
            pltpu.VMEM((NCHUNK, CH), jnp.int32),
            pltpu.VMEM((NBUF, CH, D), jnp.float32),
            pltpu.VMEM_SHARED((N, D), jnp.float32),
            pltpu.SemaphoreType.DMA((NBUF,)),
            pltpu.SemaphoreType.DMA((NBUF,)),
        ],
    )
    def prop(edges, g, zeros, out, src_v, dst_v, rows_v, acc_sh, gsems, ssems):
        c = lax.axis_index("c")
        s = lax.axis_index("s")
        wid = c * NS + s
        pltpu.sync_copy(edges.at[0, wid], src_v)
        pltpu.sync_copy(edges.at[1, wid], dst_v)
        r0 = s * RSTRIDE
        pltpu.sync_copy(zeros.at[pl.ds(r0, RLEN)], acc_sh.at[pl.ds(r0, RLEN)])
        plsc.subcore_barrier()

        def _gather(j, slot):
            return pltpu.make_async_copy(
                g.at[src_v.at[j]], rows_v.at[slot], gsems.at[slot])

        def _scatter(j, slot):
            return pltpu.make_async_copy(
                rows_v.at[slot], acc_sh.at[dst_v.at[j]], ssems.at[slot])

        for b in range(GPRE):
            _gather(b, b).start()

        def body(j, carry):
            slot = lax.rem(j, NBUF)
            _gather(j, slot).wait()
            _scatter(j, slot).start(add=True)
            jn = j + GPRE

            @pl.when(jn < NCHUNK)
            def _():
                nslot = lax.rem(jn, NBUF)

                @pl.when(jn >= NBUF)
                def _():
                    _scatter(jn - NBUF, nslot).wait()

                _gather(jn, nslot).start()

            return carry

        lax.fori_loop(0, NCHUNK, body, 0, unroll=False)
        for t in range(NBUF - GPRE):
            jd = NCHUNK - (NBUF - GPRE) + t
            _scatter(jd, jd % NBUF).wait()
        plsc.subcore_barrier()
        pltpu.sync_copy(acc_sh.at[pl.ds(r0, RLEN)], out.at[c, pl.ds(r0, RLEN)])

    return prop


@functools.cache
def _make_deg():
    @functools.partial(
        pl.kernel,
        out_type=jax.ShapeDtypeStruct((NC, N, K), jnp.float32),
        mesh=_mesh(),
        compiler_params=pltpu.CompilerParams(use_tc_tiling_on_sc=False),
        scratch_types=[
            pltpu.VMEM((NCHUNK, CH), jnp.int32),
            pltpu.VMEM((CH, K), jnp.float32),
            pltpu.VMEM_SHARED((N, K), jnp.float32),
            pltpu.SemaphoreType.DMA,
        ],
    )
    def deg(edges, ones, zeros, out, dst_v, ones_v, acc_sh, sem):
        c = lax.axis_index("c")
        s = lax.axis_index("s")
        wid = c * NS + s
        pltpu.sync_copy(edges.at[1, wid], dst_v)
        pltpu.sync_copy(ones, ones_v)
        r0 = s * RSTRIDE
        pltpu.sync_copy(zeros.at[pl.ds(r0, RLEN)], acc_sh.at[pl.ds(r0, RLEN)])
        plsc.subcore_barrier()

        def body(grp, carry):
            j0 = grp * DEGK
            for t in range(DEGK):
                pltpu.make_async_copy(
                    ones_v, acc_sh.at[dst_v.at[j0 + t]], sem).start(add=True)
            for t in range(DEGK):
                pltpu.make_async_copy(
                    ones_v, acc_sh.at[dst_v.at[j0 + t]], sem).wait()
            return carry

        lax.fori_loop(0, NCHUNK // DEGK, body, 0, unroll=False)
        plsc.subcore_barrier()
        pltpu.sync_copy(acc_sh.at[pl.ds(r0, RLEN)], out.at[c, pl.ds(r0, RLEN)])

    return deg


RB = 2000


def _u_of(degp0, degp1):
    deg = degp0[:, 0] + degp1[:, 0] + 1.0
    return lax.rsqrt(deg)


def _g1_body(x_ref, w_ref, degp_ref, g1_ref):
    u = _u_of(degp_ref[0], degp_ref[1])
    hw = jnp.dot(x_ref[...], w_ref[...], preferred_element_type=jnp.float32,
                 precision=lax.Precision.DEFAULT)
    g1_ref[...] = hw * u[:, None]


def _mid_body(s1p_ref, g1_ref, degp_ref, b1_ref, w2_ref, g2_ref):
    u = _u_of(degp_ref[0], degp_ref[1])
    h1 = u[:, None] * (s1p_ref[0] + s1p_ref[1] + g1_ref[...]) + b1_ref[...]
    h1 = jnp.maximum(h1, 0.0)
    hw2 = jnp.dot(h1, w2_ref[...], preferred_element_type=jnp.float32,
                  precision=lax.Precision.DEFAULT)
    g2_ref[...] = hw2 * u[:, None]


def _out_body(s2p_ref, g2_ref, degp_ref, b2_ref, o_ref):
    u = _u_of(degp_ref[0], degp_ref[1])
    logits = u[:, None] * (s2p_ref[0] + s2p_ref[1] + g2_ref[...]) + b2_ref[...]
    m = jnp.max(logits, axis=1, keepdims=True)
    e = jnp.exp(logits - m)
    o_ref[...] = e / jnp.sum(e, axis=1, keepdims=True)


def kernel(x, edge_index, adj, W1, b1, W2, b2):
    del adj
    er = edge_index.astype(jnp.int32).reshape(2, NW, NCHUNK, CH)
    z64 = jnp.zeros((N, HID), jnp.float32)
    z16 = jnp.zeros((N, K), jnp.float32)
    ones = jnp.ones((CH, K), jnp.float32)

    degp = _make_deg()(er, ones, z16)

    grid = (N // RB,)
    g1 = pl.pallas_call(
        _g1_body,
        grid=grid,
        in_specs=[
            pl.BlockSpec((RB, IN_CH), lambda i: (i, 0)),
            pl.BlockSpec((IN_CH, HID), lambda i: (0, 0)),
            pl.BlockSpec((NC, RB, K), lambda i: (0, i, 0)),
        ],
        out_specs=pl.BlockSpec((RB, HID), lambda i: (i, 0)),
        out_shape=jax.ShapeDtypeStruct((N, HID), jnp.float32),
    )(x, W1, degp)

    s1p = _make_prop(HID)(er, g1, z64)

    g2 = pl.pallas_call(
        _mid_body,
        grid=grid,
        in_specs=[
            pl.BlockSpec((NC, RB, HID), lambda i: (0, i, 0)),
            pl.BlockSpec((RB, HID), lambda i: (i, 0)),
            pl.BlockSpec((NC, RB, K), lambda i: (0, i, 0)),
            pl.BlockSpec((HID,), lambda i: (0,)),
            pl.BlockSpec((HID, K), lambda i: (0, 0)),
        ],
        out_specs=pl.BlockSpec((RB, K), lambda i: (i, 0)),
        out_shape=jax.ShapeDtypeStruct((N, K), jnp.float32),
    )(s1p, g1, degp, b1, W2)

    s2p = _make_prop(K)(er, g2, z16)

    out = pl.pallas_call(
        _out_body,
        grid=grid,
        in_specs=[
            pl.BlockSpec((NC, RB, K), lambda i: (0, i, 0)),
            pl.BlockSpec((RB, K), lambda i: (i, 0)),
            pl.BlockSpec((NC, RB, K), lambda i: (0, i, 0)),
            pl.BlockSpec((K,), lambda i: (0,)),
        ],
        out_specs=pl.BlockSpec((RB, K), lambda i: (i, 0)),
        out_shape=jax.ShapeDtypeStruct((N, K), jnp.float32),
    )(s2p, g2, degp, b2)
    return out


# speedup vs baseline: 52.2090x; 1.0278x over previous
import functools

import jax
import jax.numpy as jnp
from jax import lax
from jax.experimental import pallas as pl
from jax.experimental.pallas import tpu as pltpu
from jax.experimental.pallas import tpu_sc as plsc

N = 10000
E = 320000
IN_CH = 128
HID = 64
K = 16

NC, NS = 2, 16
NW = NC * NS
EPT = E // NW
CH = 125
NCHUNK = EPT // CH
NBUF = 8
GPRE = 4
DEGK = 8
RSTRIDE = 624
RLEN = 640
ZR = 160

@functools.cache
def _mesh():
    return plsc.VectorSubcoreMesh(
        core_axis_name="c", subcore_axis_name="s", num_cores=NC, num_subcores=NS)


@functools.cache
def _make_prop(D, spmem_table):
    table_scratch = (
        [pltpu.VMEM_SHARED((N, D), jnp.float32)] if spmem_table else [])

    @functools.partial(
        pl.kernel,
        out_type=jax.ShapeDtypeStruct((NC, N, D), jnp.float32),
        mesh=_mesh(),
        compiler_params=pltpu.CompilerParams(use_tc_tiling_on_sc=False),
        scratch_types=[
            pltpu.VMEM((NCHUNK, CH), jnp.int32),
            pltpu.VMEM((NCHUNK, CH), jnp.int32),
            pltpu.VMEM((NBUF, CH, D), jnp.float32),
            pltpu.VMEM_SHARED((N, D), jnp.float32),
        ] + table_scratch + [
            pltpu.SemaphoreType.DMA((NBUF,)),
            pltpu.SemaphoreType.DMA((NBUF,)),
        ],
    )
    def prop(edges, g, zeros, out, src_v, dst_v, rows_v, acc_sh, *rest):
        if spmem_table:
            g_sh, gsems, ssems = rest
        else:
            gsems, ssems = rest
            g_sh = g
        c = lax.axis_index("c")
        s = lax.axis_index("s")
        wid = c * NS + s
        pltpu.sync_copy(edges.at[0, wid], src_v)
        pltpu.sync_copy(edges.at[1, wid], dst_v)
        r0 = s * RSTRIDE
        pltpu.sync_copy(zeros.at[pl.ds(r0, RLEN)], acc_sh.at[pl.ds(r0, RLEN)])
        if spmem_table:
            pltpu.sync_copy(g.at[pl.ds(r0, RLEN)], g_sh.at[pl.ds(r0, RLEN)])
        plsc.subcore_barrier()

        def _gather(j, slot):
            return pltpu.make_async_copy(
                g_sh.at[src_v.at[j]], rows_v.at[slot], gsems.at[slot])

        def _scatter(j, slot):
            return pltpu.make_async_copy(
                rows_v.at[slot], acc_sh.at[dst_v.at[j]], ssems.at[slot])

        for b in range(GPRE):
            _gather(b, b).start()

        def body(j, carry):
            slot = lax.rem(j, NBUF)
            _gather(j, slot).wait()
            _scatter(j, slot).start(add=True)
            jn = j + GPRE

            @pl.when(jn < NCHUNK)
            def _():
                nslot = lax.rem(jn, NBUF)

                @pl.when(jn >= NBUF)
                def _():
                    _scatter(jn - NBUF, nslot).wait()

                _gather(jn, nslot).start()

            return carry

        lax.fori_loop(0, NCHUNK, body, 0, unroll=False)
        for t in range(NBUF - GPRE):
            jd = NCHUNK - (NBUF - GPRE) + t
            _scatter(jd, jd % NBUF).wait()
        plsc.subcore_barrier()
        pltpu.sync_copy(acc_sh.at[pl.ds(r0, RLEN)], out.at[c, pl.ds(r0, RLEN)])

    return prop


@functools.cache
def _make_deg():
    @functools.partial(
        pl.kernel,
        out_type=jax.ShapeDtypeStruct((NC, N, K), jnp.float32),
        mesh=_mesh(),
        compiler_params=pltpu.CompilerParams(use_tc_tiling_on_sc=False),
        scratch_types=[
            pltpu.VMEM((NCHUNK, CH), jnp.int32),
            pltpu.VMEM((CH, K), jnp.float32),
            pltpu.VMEM_SHARED((N, K), jnp.float32),
            pltpu.SemaphoreType.DMA,
        ],
    )
    def deg(edges, ones, zeros, out, dst_v, ones_v, acc_sh, sem):
        c = lax.axis_index("c")
        s = lax.axis_index("s")
        wid = c * NS + s
        pltpu.sync_copy(edges.at[1, wid], dst_v)
        pltpu.sync_copy(ones, ones_v)
        r0 = s * RSTRIDE
        pltpu.sync_copy(zeros.at[pl.ds(r0, RLEN)], acc_sh.at[pl.ds(r0, RLEN)])
        plsc.subcore_barrier()

        def body(grp, carry):
            j0 = grp * DEGK
            for t in range(DEGK):
                pltpu.make_async_copy(
                    ones_v, acc_sh.at[dst_v.at[j0 + t]], sem).start(add=True)
            for t in range(DEGK):
                pltpu.make_async_copy(
                    ones_v, acc_sh.at[dst_v.at[j0 + t]], sem).wait()
            return carry

        lax.fori_loop(0, NCHUNK // DEGK, body, 0, unroll=False)
        plsc.subcore_barrier()
        pltpu.sync_copy(acc_sh.at[pl.ds(r0, RLEN)], out.at[c, pl.ds(r0, RLEN)])

    return deg


RB = 2000


def _u_of(degp0, degp1):
    deg = degp0[:, 0] + degp1[:, 0] + 1.0
    return lax.rsqrt(deg)


def _g1_body(x_ref, w_ref, degp_ref, g1_ref):
    u = _u_of(degp_ref[0], degp_ref[1])
    hw = jnp.dot(x_ref[...], w_ref[...], preferred_element_type=jnp.float32,
                 precision=lax.Precision.DEFAULT)
    g1_ref[...] = hw * u[:, None]


def _mid_body(s1p_ref, g1_ref, degp_ref, b1_ref, w2_ref, g2_ref):
    u = _u_of(degp_ref[0], degp_ref[1])
    h1 = u[:, None] * (s1p_ref[0] + s1p_ref[1] + g1_ref[...]) + b1_ref[...]
    h1 = jnp.maximum(h1, 0.0)
    hw2 = jnp.dot(h1, w2_ref[...], preferred_element_type=jnp.float32,
                  precision=lax.Precision.DEFAULT)
    g2_ref[...] = hw2 * u[:, None]


def _out_body(s2p_ref, g2_ref, degp_ref, b2_ref, o_ref):
    u = _u_of(degp_ref[0], degp_ref[1])
    logits = u[:, None] * (s2p_ref[0] + s2p_ref[1] + g2_ref[...]) + b2_ref[...]
    m = jnp.max(logits, axis=1, keepdims=True)
    e = jnp.exp(logits - m)
    o_ref[...] = e / jnp.sum(e, axis=1, keepdims=True)


def kernel(x, edge_index, adj, W1, b1, W2, b2):
    del adj
    er = edge_index.astype(jnp.int32).reshape(2, NW, NCHUNK, CH)
    z16 = jnp.zeros((N, K), jnp.float32)
    z64 = jnp.zeros((N, HID), jnp.float32)
    ones = jnp.ones((CH, K), jnp.float32)

    degp = _make_deg()(er, ones, z16)

    grid = (N // RB,)
    g1 = pl.pallas_call(
        _g1_body,
        grid=grid,
        in_specs=[
            pl.BlockSpec((RB, IN_CH), lambda i: (i, 0)),
            pl.BlockSpec((IN_CH, HID), lambda i: (0, 0)),
            pl.BlockSpec((NC, RB, K), lambda i: (0, i, 0)),
        ],
        out_specs=pl.BlockSpec((RB, HID), lambda i: (i, 0)),
        out_shape=jax.ShapeDtypeStruct((N, HID), jnp.float32),
    )(x, W1, degp)

    s1p = _make_prop(HID, False)(er, g1, z64)

    g2 = pl.pallas_call(
        _mid_body,
        grid=grid,
        in_specs=[
            pl.BlockSpec((NC, RB, HID), lambda i: (0, i, 0)),
            pl.BlockSpec((RB, HID), lambda i: (i, 0)),
            pl.BlockSpec((NC, RB, K), lambda i: (0, i, 0)),
            pl.BlockSpec((HID,), lambda i: (0,)),
            pl.BlockSpec((HID, K), lambda i: (0, 0)),
        ],
        out_specs=pl.BlockSpec((RB, K), lambda i: (i, 0)),
        out_shape=jax.ShapeDtypeStruct((N, K), jnp.float32),
    )(s1p, g1, degp, b1, W2)

    s2p = _make_prop(K, True)(er, g2, z16)

    out = pl.pallas_call(
        _out_body,
        grid=grid,
        in_specs=[
            pl.BlockSpec((NC, RB, K), lambda i: (0, i, 0)),
            pl.BlockSpec((RB, K), lambda i: (i, 0)),
            pl.BlockSpec((NC, RB, K), lambda i: (0, i, 0)),
            pl.BlockSpec((K,), lambda i: (0,)),
        ],
        out_specs=pl.BlockSpec((RB, K), lambda i: (i, 0)),
        out_shape=jax.ShapeDtypeStruct((N, K), jnp.float32),
    )(s2p, g2, degp, b2)
    return out
